# baseline (speedup 1.0000x reference)
import jax
import jax.numpy as jnp
from jax.experimental import pallas as pl
from jax.experimental.pallas import tpu as pltpu


def _mm_bias_kernel(x_ref, w_ref, b_ref, o_ref):
    acc = jnp.dot(x_ref[...], w_ref[...], preferred_element_type=jnp.float32)
    o_ref[...] = acc + b_ref[...]


def kernel(x, weight, bias):
    p = 16
    B, C, H, W = x.shape
    Hp, Wp = H // p, W // p
    N = Hp * Wp
    K = C * p * p
    E = weight.shape[1]
    M = B * N

    xr = x.reshape(B, C, Hp, p, Wp, p)
    xr = jnp.transpose(xr, (0, 2, 4, 3, 5, 1)).reshape(M, K)
    xb = xr.astype(jnp.bfloat16)
    wb = weight.astype(jnp.bfloat16)
    b2 = bias.reshape(1, E).astype(jnp.float32)

    tm = 448
    out = pl.pallas_call(
        _mm_bias_kernel,
        out_shape=jax.ShapeDtypeStruct((M, E), jnp.float32),
        grid=(M // tm,),
        in_specs=[
            pl.BlockSpec((tm, K), lambda i: (i, 0)),
            pl.BlockSpec((K, E), lambda i: (0, 0)),
            pl.BlockSpec((1, E), lambda i: (0, 0)),
        ],
        out_specs=pl.BlockSpec((tm, E), lambda i: (i, 0)),
        compiler_params=pltpu.CompilerParams(
            dimension_semantics=("parallel",),
        ),
    )(xb, wb, b2)
    return out.reshape(B, N, E)


# speedup vs baseline: 6.5575x; 6.5575x over previous
import jax
import jax.numpy as jnp
from jax.experimental import pallas as pl
from jax.experimental.pallas import tpu as pltpu


def _mm_bias_kernel(x_ref, w_ref, b_ref, o_ref):
    xb = x_ref[...].astype(jnp.bfloat16)
    acc = jnp.dot(xb, w_ref[...], preferred_element_type=jnp.float32)
    o_ref[...] = acc + b_ref[...]


def kernel(x, weight, bias):
    p = 16
    B, C, H, W = x.shape
    Hp, Wp = H // p, W // p
    N = Hp * Wp
    K = C * p * p
    E = weight.shape[1]
    M = B * N

    xr = x.reshape(B, C, Hp, p, Wp, p)
    xb = jnp.transpose(xr, (0, 2, 4, 3, 5, 1)).reshape(M, K)
    wb = weight.astype(jnp.bfloat16)
    b2 = bias.reshape(1, E).astype(jnp.float32)

    tm = 448
    out = pl.pallas_call(
        _mm_bias_kernel,
        out_shape=jax.ShapeDtypeStruct((M, E), jnp.float32),
        grid=(M // tm,),
        in_specs=[
            pl.BlockSpec((tm, K), lambda i: (i, 0)),
            pl.BlockSpec((K, E), lambda i: (0, 0)),
            pl.BlockSpec((1, E), lambda i: (0, 0)),
        ],
        out_specs=pl.BlockSpec((tm, E), lambda i: (i, 0)),
        compiler_params=pltpu.CompilerParams(
            dimension_semantics=("parallel",),
        ),
    )(xb, wb, b2)
    return out.reshape(B, N, E)
